# Initial kernel scaffold; baseline (speedup 1.0000x reference)
import functools

import jax
import jax.numpy as jnp
from jax import lax
from jax.experimental import pallas as pl
from jax.experimental.pallas import tpu as pltpu
from jax.experimental.pallas import tpu_sc as plsc

N = 10000
E = 320000
D = 128

NC = 2
NS = 16
NW = NC * NS
L = 16

EPW = E // NW
K = 80
NB = EPW // K
RPT = N // NS
ZR = 125


def _make_sc_kernel():
    mesh = plsc.VectorSubcoreMesh(
        core_axis_name="c", subcore_axis_name="s",
        num_cores=NC, num_subcores=NS)

    @functools.partial(
        pl.kernel,
        out_type=jax.ShapeDtypeStruct((NC, N, D), jnp.float32),
        mesh=mesh,
        scratch_types=[
            pltpu.VMEM((K,), jnp.int32),
            pltpu.VMEM((K,), jnp.int32),
            pltpu.VMEM((K,), jnp.float32),
            pltpu.VMEM((K, D), jnp.float32),
            pltpu.VMEM((ZR, D), jnp.float32),
            pltpu.VMEM_SHARED((N, D), jnp.float32),
            pltpu.SemaphoreType.DMA,
        ],
    )
    def spmm(x_hbm, col_hbm, row_hbm, w_hbm, out_hbm,
             colv, rowv, wv, rows, zbuf, acc, sem):
        cid = lax.axis_index("c")
        sid = lax.axis_index("s")
        wid = cid * NS + sid

        def zrow(r, _):
            for j in range(D // L):
                zbuf[r, pl.ds(j * L, L)] = jnp.zeros((L,), jnp.float32)
            return 0
        lax.fori_loop(0, ZR, zrow, 0)
        for j in range(RPT // ZR):
            pltpu.sync_copy(zbuf, acc.at[pl.ds(sid * RPT + j * ZR, ZR)])
        plsc.subcore_barrier()

        base = wid * EPW

        def batch(b, _):
            off = base + b * K
            pltpu.sync_copy(col_hbm.at[pl.ds(off, K)], colv)
            pltpu.sync_copy(row_hbm.at[pl.ds(off, K)], rowv)
            pltpu.sync_copy(w_hbm.at[pl.ds(off, K)], wv)
            pltpu.async_copy(x_hbm.at[colv], rows, sem).wait()

            def group(g, _):
                for e in range(L):
                    idx = g * L + e
                    wb = plsc.load_gather(
                        wv, [jnp.full((L,), idx, jnp.int32)])
                    for j in range(D // L):
                        rows[idx, pl.ds(j * L, L)] = (
                            rows[idx, pl.ds(j * L, L)] * wb)
                return 0
            lax.fori_loop(0, K // L, group, 0)

            pltpu.sync_copy(rows, acc.at[rowv], add=True)
            return 0
        lax.fori_loop(0, NB, batch, 0)

        plsc.subcore_barrier()

        for j in range(RPT // ZR):
            r0 = sid * RPT + j * ZR
            pltpu.sync_copy(acc.at[pl.ds(r0, ZR)],
                            out_hbm.at[cid, pl.ds(r0, ZR)])

    return spmm


_sc_spmm = _make_sc_kernel()


def _add_body(a_ref, b_ref, o_ref):
    o_ref[...] = a_ref[...] + b_ref[...]


def _combine(p0, p1):
    blk = 1000
    return pl.pallas_call(
        _add_body,
        out_shape=jax.ShapeDtypeStruct((N, D), jnp.float32),
        grid=(N // blk,),
        in_specs=[pl.BlockSpec((blk, D), lambda i: (i, 0)),
                  pl.BlockSpec((blk, D), lambda i: (i, 0))],
        out_specs=pl.BlockSpec((blk, D), lambda i: (i, 0)),
    )(p0, p1)


def kernel(t, x, edge_index, edge_weight):
    row = edge_index[0].astype(jnp.int32)
    col = edge_index[1].astype(jnp.int32)
    w = edge_weight.astype(jnp.float32)
    partials = _sc_spmm(x.astype(jnp.float32), col, row, w)
    return _combine(partials[0], partials[1])


# speedup vs baseline: 4.1209x; 4.1209x over previous
import functools

import jax
import jax.numpy as jnp
from jax import lax
from jax.experimental import pallas as pl
from jax.experimental.pallas import tpu as pltpu
from jax.experimental.pallas import tpu_sc as plsc

N = 10000
E = 320000
D = 128

NC = 2
NS = 16
NW = NC * NS
L = 16

EPW = E // NW
K = 80
NB = EPW // K
RPT = 624
ZR = 208
TAIL = N - NS * RPT


def _make_sc_kernel():
    mesh = plsc.VectorSubcoreMesh(
        core_axis_name="c", subcore_axis_name="s",
        num_cores=NC, num_subcores=NS)

    @functools.partial(
        pl.kernel,
        out_type=jax.ShapeDtypeStruct((NC, N, D), jnp.float32),
        mesh=mesh,
        scratch_types=[
            pltpu.VMEM((K,), jnp.int32),
            pltpu.VMEM((K,), jnp.int32),
            pltpu.VMEM((K,), jnp.float32),
            pltpu.VMEM((K, D), jnp.float32),
            pltpu.VMEM((ZR, D), jnp.float32),
            pltpu.VMEM_SHARED((N, D), jnp.float32),
            pltpu.SemaphoreType.DMA,
        ],
    )
    def spmm(x_hbm, col_hbm, row_hbm, w_hbm, out_hbm,
             colv, rowv, wv, rows, zbuf, acc, sem):
        cid = lax.axis_index("c")
        sid = lax.axis_index("s")
        wid = cid * NS + sid

        def zrow(r, _):
            for j in range(D // L):
                zbuf[r, pl.ds(j * L, L)] = jnp.zeros((L,), jnp.float32)
            return 0
        lax.fori_loop(0, ZR, zrow, 0)
        for j in range(RPT // ZR):
            pltpu.sync_copy(zbuf, acc.at[pl.ds(sid * RPT + j * ZR, ZR)])

        @pl.when(sid == NS - 1)
        def _zero_tail():
            pltpu.sync_copy(zbuf.at[pl.ds(0, TAIL)],
                            acc.at[pl.ds(NS * RPT, TAIL)])

        plsc.subcore_barrier()

        base = wid * EPW

        def batch(b, _):
            off = base + b * K
            pltpu.sync_copy(col_hbm.at[pl.ds(off, K)], colv)
            pltpu.sync_copy(row_hbm.at[pl.ds(off, K)], rowv)
            pltpu.sync_copy(w_hbm.at[pl.ds(off, K)], wv)
            pltpu.async_copy(x_hbm.at[colv], rows, sem).wait()

            def group(g, _):
                gvec = wv[pl.ds(g * L, L)]
                for e in range(L):
                    idx = g * L + e
                    wb = gvec.at[jnp.full((L,), e, jnp.int32)].get(
                        mode="promise_in_bounds")
                    for j in range(D // L):
                        rows[idx, pl.ds(j * L, L)] = (
                            rows[idx, pl.ds(j * L, L)] * wb)
                return 0
            lax.fori_loop(0, K // L, group, 0)

            pltpu.sync_copy(rows, acc.at[rowv], add=True)
            return 0
        lax.fori_loop(0, NB, batch, 0)

        plsc.subcore_barrier()

        for j in range(RPT // ZR):
            r0 = sid * RPT + j * ZR
            pltpu.sync_copy(acc.at[pl.ds(r0, ZR)],
                            out_hbm.at[cid, pl.ds(r0, ZR)])

        @pl.when(sid == NS - 1)
        def _copy_tail():
            pltpu.sync_copy(acc.at[pl.ds(NS * RPT, TAIL)],
                            out_hbm.at[cid, pl.ds(NS * RPT, TAIL)])

    return spmm


_sc_spmm = _make_sc_kernel()


def _add_body(a_ref, b_ref, o_ref):
    o_ref[...] = a_ref[...] + b_ref[...]


def _combine(p0, p1):
    blk = 1000
    return pl.pallas_call(
        _add_body,
        out_shape=jax.ShapeDtypeStruct((N, D), jnp.float32),
        grid=(N // blk,),
        in_specs=[pl.BlockSpec((blk, D), lambda i: (i, 0)),
                  pl.BlockSpec((blk, D), lambda i: (i, 0))],
        out_specs=pl.BlockSpec((blk, D), lambda i: (i, 0)),
    )(p0, p1)


def kernel(t, x, edge_index, edge_weight):
    row = edge_index[0].astype(jnp.int32)
    col = edge_index[1].astype(jnp.int32)
    w = edge_weight.astype(jnp.float32)
    partials = _sc_spmm(x.astype(jnp.float32), col, row, w)
    return _combine(partials[0], partials[1])


# speedup vs baseline: 10.8649x; 2.6365x over previous
import functools

import jax
import jax.numpy as jnp
from jax import lax
from jax.experimental import pallas as pl
from jax.experimental.pallas import tpu as pltpu
from jax.experimental.pallas import tpu_sc as plsc

N = 10000
E = 320000
D = 128

NC = 2
NS = 16
NW = NC * NS
L = 16

EPW = E // NW
K = 80
NB = EPW // K
NBUF = 4
PD = 1
RPT = 624
ZR = 208
TAIL = N - NS * RPT


def _make_sc_kernel():
    mesh = plsc.VectorSubcoreMesh(
        core_axis_name="c", subcore_axis_name="s",
        num_cores=NC, num_subcores=NS)

    @functools.partial(
        pl.kernel,
        out_type=jax.ShapeDtypeStruct((NC, N, D), jnp.float32),
        mesh=mesh,
        scratch_types=[
            [pltpu.VMEM((K, D), jnp.float32) for _ in range(NBUF)],
            [pltpu.VMEM((K,), jnp.int32) for _ in range(NBUF)],
            [pltpu.VMEM((K,), jnp.int32) for _ in range(NBUF)],
            [pltpu.VMEM((K,), jnp.float32) for _ in range(NBUF)],
            pltpu.VMEM_SHARED((N, D), jnp.float32),
            [pltpu.SemaphoreType.DMA for _ in range(NBUF)],
            [pltpu.SemaphoreType.DMA for _ in range(NBUF)],
            [pltpu.SemaphoreType.DMA for _ in range(NBUF)],
        ],
    )
    def spmm(x_hbm, col_hbm, row_hbm, w_hbm, z_hbm, out_hbm,
             rows, cidx, ridx, wvb, acc, isem, gsem, ssem):
        cid = lax.axis_index("c")
        sid = lax.axis_index("s")
        wid = cid * NS + sid
        base = wid * EPW

        for j in range(RPT // ZR):
            pltpu.sync_copy(z_hbm, acc.at[pl.ds(sid * RPT + j * ZR, ZR)])

        @pl.when(sid == NS - 1)
        def _zero_tail():
            pltpu.sync_copy(z_hbm.at[pl.ds(0, TAIL)],
                            acc.at[pl.ds(NS * RPT, TAIL)])

        plsc.subcore_barrier()

        def fire_idx(b, buf):
            off = base + b * K
            pltpu.async_copy(col_hbm.at[pl.ds(off, K)], cidx[buf], isem[buf])
            pltpu.async_copy(row_hbm.at[pl.ds(off, K)], ridx[buf], isem[buf])
            pltpu.async_copy(w_hbm.at[pl.ds(off, K)], wvb[buf], isem[buf])

        def wait_idx(buf):
            pltpu.make_async_copy(
                col_hbm.at[pl.ds(0, K)], cidx[buf], isem[buf]).wait()
            pltpu.make_async_copy(
                row_hbm.at[pl.ds(0, K)], ridx[buf], isem[buf]).wait()
            pltpu.make_async_copy(
                w_hbm.at[pl.ds(0, K)], wvb[buf], isem[buf]).wait()

        def fire_gather(buf):
            pltpu.async_copy(x_hbm.at[cidx[buf]], rows[buf], gsem[buf])

        def wait_gather(buf):
            pltpu.make_async_copy(
                x_hbm.at[cidx[buf]], rows[buf], gsem[buf]).wait()

        def fire_scatter(buf):
            pltpu.async_copy(rows[buf], acc.at[ridx[buf]], ssem[buf],
                             add=True)

        def wait_scatter(buf):
            pltpu.make_async_copy(
                rows[buf], acc.at[ridx[buf]], ssem[buf]).wait()

        for p in range(PD + 1):
            fire_idx(p, p)
        for p in range(PD):
            wait_idx(p)
            fire_gather(p)

        def outer(ob, _):
            b0 = ob * NBUF
            for ph in range(NBUF):
                bb = b0 + ph

                ibuf = (ph + PD + 1) % NBUF

                @pl.when(bb + PD + 1 < NB)
                def _prefetch_idx():
                    @pl.when(bb >= NBUF - PD - 1)
                    def _drain_scatter():
                        wait_scatter(ibuf)
                    fire_idx(bb + PD + 1, ibuf)

                gbuf = (ph + PD) % NBUF

                @pl.when(bb + PD < NB)
                def _prefetch_gather():
                    wait_idx(gbuf)
                    fire_gather(gbuf)

                @pl.when(bb < NB)
                def _process():
                    wait_gather(ph)

                    def group(gr, _):
                        gvec = wvb[ph][pl.ds(gr * L, L)]
                        for e in range(L):
                            idx = gr * L + e
                            wb = gvec.at[jnp.full((L,), e, jnp.int32)].get(
                                mode="promise_in_bounds")
                            for j in range(D // L):
                                rows[ph][idx, pl.ds(j * L, L)] = (
                                    rows[ph][idx, pl.ds(j * L, L)] * wb)
                        return 0
                    lax.fori_loop(0, K // L, group, 0)

                    fire_scatter(ph)
            return 0
        lax.fori_loop(0, pl.cdiv(NB, NBUF), outer, 0)

        for ph in range(NBUF):
            wait_scatter(ph)

        plsc.subcore_barrier()

        for j in range(RPT // ZR):
            r0 = sid * RPT + j * ZR
            pltpu.sync_copy(acc.at[pl.ds(r0, ZR)],
                            out_hbm.at[cid, pl.ds(r0, ZR)])

        @pl.when(sid == NS - 1)
        def _copy_tail():
            pltpu.sync_copy(acc.at[pl.ds(NS * RPT, TAIL)],
                            out_hbm.at[cid, pl.ds(NS * RPT, TAIL)])

    return spmm


_sc_spmm = _make_sc_kernel()


def _add_body(a_ref, b_ref, o_ref):
    o_ref[...] = a_ref[...] + b_ref[...]


def _combine(p0, p1):
    blk = 1000
    return pl.pallas_call(
        _add_body,
        out_shape=jax.ShapeDtypeStruct((N, D), jnp.float32),
        grid=(N // blk,),
        in_specs=[pl.BlockSpec((blk, D), lambda i: (i, 0)),
                  pl.BlockSpec((blk, D), lambda i: (i, 0))],
        out_specs=pl.BlockSpec((blk, D), lambda i: (i, 0)),
    )(p0, p1)


def kernel(t, x, edge_index, edge_weight):
    row = edge_index[0].astype(jnp.int32)
    col = edge_index[1].astype(jnp.int32)
    w = edge_weight.astype(jnp.float32)
    z = jnp.zeros((ZR, D), jnp.float32)
    partials = _sc_spmm(x.astype(jnp.float32), col, row, w, z)
    return _combine(partials[0], partials[1])
